# baseline (speedup 1.0000x reference)
import jax
import jax.numpy as jnp
from jax import lax
from jax.experimental import pallas as pl
from jax.experimental.pallas import tpu as pltpu
from jax.experimental.pallas import tpu_sc as plsc

N_NODES = 100000
NUM_SPECIES = 128
EMBED_DIM = 128
CHUNK = 128
FLAT = CHUNK * EMBED_DIM
NUM_CORES = 2
NUM_SUBCORES = 16
NUM_WORKERS = NUM_CORES * NUM_SUBCORES
NUM_CHUNKS = -(-N_NODES // CHUNK)
TRIPS = -(-NUM_CHUNKS // NUM_WORKERS)
LAST_START = N_NODES - CHUNK
NBUF = 3
C_UNITS = 5
S_UNITS = TRIPS - C_UNITS
C_GROUPS = C_UNITS * 8
LANES = 16


def _gather_body(idx_hbm, w_hbm, out_hbm,
                 idx_v, r0, r1, r2, idx_c, cb0, cb1, w_loc, w_sh,
                 sem_i, sem_g, sem_s, sem_ci, sem_cs):
    c = lax.axis_index("c")
    s = lax.axis_index("s")
    wid = s * NUM_CORES + c
    rows = [r0, r1, r2]
    cbs = [cb0, cb1]
    pltpu.sync_copy(w_hbm, w_sh)
    pltpu.sync_copy(w_hbm, w_loc)

    def start_of(u):
        return jnp.minimum((wid + u * NUM_WORKERS) * CHUNK, LAST_START)

    def load_idx(j):
        b = j % NBUF
        return pltpu.async_copy(
            idx_hbm.at[pl.ds(start_of(j), CHUNK)], idx_v.at[b], sem_i.at[b])

    def gather(j):
        b = j % NBUF
        return pltpu.async_copy(w_sh.at[idx_v.at[b]], rows[b], sem_g.at[b])

    def store(j):
        b = j % NBUF
        return pltpu.async_copy(
            rows[b], out_hbm.at[pl.ds(start_of(j), CHUNK)], sem_s.at[b])

    h_ci = [pltpu.async_copy(
        idx_hbm.at[pl.ds(start_of(S_UNITS + k), CHUNK)],
        idx_c.at[pl.ds(k * CHUNK, CHUNK)], sem_ci) for k in range(C_UNITS)]

    iota = jax.lax.broadcasted_iota(jnp.int32, (LANES,), 0)

    def compute_group(t):
        k = t // 8
        g = t % 8
        b = k % 2
        cb = cbs[b]
        nidx = idx_c[pl.ds(t * LANES, LANES)]
        orow = g * LANES + iota
        zero = iota * 0

        def col_block(q, carry):
            c1 = zero + q * LANES
            for cc in range(LANES):
                v = plsc.load_gather(w_loc, [nidx, c1 + cc])
                plsc.store_scatter(cb, [orow, c1 + cc], v)
            return carry

        lax.fori_loop(0, 8, col_block, 0)

    def cstore(k):
        b = k % 2
        return pltpu.async_copy(
            cbs[b],
            out_hbm.at[pl.ds(start_of(S_UNITS + k), CHUNK)],
            sem_cs.at[b])

    h_idx = [None] * S_UNITS
    h_s = [None] * S_UNITS
    hc_s = [None] * C_UNITS

    for j in range(min(NBUF, S_UNITS)):
        h_idx[j] = load_idx(j)
    for h in h_ci:
        h.wait()

    ct = 0
    for j in range(S_UNITS):
        h_idx[j].wait()
        if j >= NBUF:
            h_s[j - NBUF].wait()
        g = gather(j)
        for _ in range(2):
            if ct < C_GROUPS:
                k, gg = ct // 8, ct % 8
                if gg == 0:
                    if k >= 2:
                        hc_s[k - 2].wait()
                compute_group(ct)
                if gg == 7:
                    hc_s[k] = cstore(k)
                ct += 1
        g.wait()
        if j + NBUF < S_UNITS:
            h_idx[j + NBUF] = load_idx(j + NBUF)
        h_s[j] = store(j)
    while ct < C_GROUPS:
        k, gg = ct // 8, ct % 8
        if gg == 0 and k >= 2:
            hc_s[k - 2].wait()
        compute_group(ct)
        if gg == 7:
            hc_s[k] = cstore(k)
        ct += 1
    for j in range(max(0, S_UNITS - NBUF), S_UNITS):
        h_s[j].wait()
    for k in range(max(0, C_UNITS - 2), C_UNITS):
        hc_s[k].wait()


@jax.jit
def _embed(node_specie, w):
    mesh = plsc.VectorSubcoreMesh(
        core_axis_name="c", subcore_axis_name="s",
        num_cores=NUM_CORES, num_subcores=NUM_SUBCORES)
    return pl.kernel(
        _gather_body,
        out_type=jax.ShapeDtypeStruct((N_NODES, EMBED_DIM), jnp.float32),
        mesh=mesh,
        compiler_params=pltpu.CompilerParams(needs_layout_passes=False),
        scratch_types=[
            pltpu.VMEM((NBUF, CHUNK), jnp.int32),
            pltpu.VMEM((CHUNK, EMBED_DIM), jnp.float32),
            pltpu.VMEM((CHUNK, EMBED_DIM), jnp.float32),
            pltpu.VMEM((CHUNK, EMBED_DIM), jnp.float32),
            pltpu.VMEM((C_UNITS * CHUNK,), jnp.int32),
            pltpu.VMEM((CHUNK, EMBED_DIM), jnp.float32),
            pltpu.VMEM((CHUNK, EMBED_DIM), jnp.float32),
            pltpu.VMEM((NUM_SPECIES, EMBED_DIM), jnp.float32),
            pltpu.VMEM_SHARED((NUM_SPECIES, EMBED_DIM), jnp.float32),
            pltpu.SemaphoreType.DMA((NBUF,)),
            pltpu.SemaphoreType.DMA((NBUF,)),
            pltpu.SemaphoreType.DMA((NBUF,)),
            pltpu.SemaphoreType.DMA,
            pltpu.SemaphoreType.DMA((2,)),
        ],
    )(node_specie, w)


def kernel(node_specie, w):
    ns = node_specie.astype(jnp.int32)
    return _embed(ns, w)


# speedup vs baseline: 2.9104x; 2.9104x over previous
import jax
import jax.numpy as jnp
from jax import lax
from jax.experimental import pallas as pl
from jax.experimental.pallas import tpu as pltpu
from jax.experimental.pallas import tpu_sc as plsc

N_NODES = 100000
NUM_SPECIES = 128
EMBED_DIM = 128
SUB = 128
SUBS = 1
CHUNK = SUB * SUBS
NUM_CORES = 2
NUM_SUBCORES = 16
NUM_WORKERS = NUM_CORES * NUM_SUBCORES
NUM_CHUNKS = -(-N_NODES // CHUNK)
TRIPS = -(-NUM_CHUNKS // NUM_WORKERS)
LAST_START = N_NODES - CHUNK
NBUF = 3


def _gather_body(idx_hbm, w_hbm, out_hbm,
                 idx_v, rows_v, w_sh, sem_i, sem_g, sem_s):
    c = lax.axis_index("c")
    s = lax.axis_index("s")
    wid = s * NUM_CORES + c
    @pl.when(s == 0)
    def _stage():
        pltpu.sync_copy(w_hbm, w_sh)
    plsc.subcore_barrier()

    def start_of(j):
        return jnp.minimum((wid + j * NUM_WORKERS) * CHUNK, LAST_START)

    def load_idx(j):
        b = j % NBUF
        return [pltpu.async_copy(
            idx_hbm.at[pl.ds(start_of(j) + h * SUB, SUB)],
            idx_v.at[b, h], sem_i.at[b]) for h in range(SUBS)]

    def gather(j, h):
        b = j % NBUF
        return pltpu.async_copy(
            w_sh.at[idx_v.at[b, h]],
            rows_v.at[b, pl.ds(h * SUB, SUB)],
            sem_g.at[b])

    def store(j):
        b = j % NBUF
        return pltpu.async_copy(
            rows_v.at[b], out_hbm.at[pl.ds(start_of(j), CHUNK)], sem_s.at[b])

    h_idx = [None] * TRIPS
    h_s = [None] * TRIPS

    for j in range(min(NBUF, TRIPS)):
        h_idx[j] = load_idx(j)
    for j in range(TRIPS):
        for h in h_idx[j]:
            h.wait()
        if j >= NBUF:
            h_s[j - NBUF].wait()
        hg = [gather(j, h) for h in range(SUBS)]
        for g in hg:
            g.wait()
        if j + NBUF < TRIPS:
            h_idx[j + NBUF] = load_idx(j + NBUF)
        h_s[j] = store(j)
    for j in range(max(0, TRIPS - NBUF), TRIPS):
        h_s[j].wait()


@jax.jit
def _embed(node_specie, w):
    mesh = plsc.VectorSubcoreMesh(
        core_axis_name="c", subcore_axis_name="s",
        num_cores=NUM_CORES, num_subcores=NUM_SUBCORES)
    return pl.kernel(
        _gather_body,
        out_type=jax.ShapeDtypeStruct((N_NODES, EMBED_DIM), jnp.float32),
        mesh=mesh,
        scratch_types=[
            pltpu.VMEM((NBUF, SUBS, SUB), jnp.int32),
            pltpu.VMEM((NBUF, CHUNK, EMBED_DIM), jnp.float32),
            pltpu.VMEM_SHARED((NUM_SPECIES, EMBED_DIM), jnp.float32),
            pltpu.SemaphoreType.DMA((NBUF,)),
            pltpu.SemaphoreType.DMA((NBUF,)),
            pltpu.SemaphoreType.DMA((NBUF,)),
        ],
    )(node_specie, w)


def kernel(node_specie, w):
    return _embed(node_specie.astype(jnp.int32), w)
